# baseline (speedup 1.0000x reference)
import functools

import jax
import jax.numpy as jnp
from jax.experimental import pallas as pl
from jax.experimental.pallas import tpu as pltpu

_BLOCK_BYTES_TARGET = 1 * 1024 * 1024
_VMEM_BYTES = 64 * 1024 * 1024

_DN_T = (((1,), (1,)), ((), ()))


def _se_block(x_ref, w1_ref, b1_ref, w2_ref, b2_ref, o_ref, *, inv_hw):
    x = x_ref[...]
    pooled = jnp.sum(jnp.sum(x.astype(jnp.float32), axis=3), axis=2) * inv_hw
    h = jax.lax.dot_general(pooled, w1_ref[...], _DN_T,
                            preferred_element_type=jnp.float32)
    h = jnp.maximum(h + b1_ref[...], 0.0)
    g = jax.lax.dot_general(h, w2_ref[...], _DN_T,
                            preferred_element_type=jnp.float32)
    g = jax.nn.sigmoid(g + b2_ref[...])
    o_ref[...] = x * g[:, :, None, None].astype(x.dtype)


def _group_size(batch, batch_item_bytes):
    cap = max(1, _BLOCK_BYTES_TARGET // max(batch_item_bytes, 1))
    nb = 1
    for d in range(1, min(batch, cap) + 1):
        if batch % d == 0:
            nb = d
    return nb


def kernel(x_nchw, w1, b1, w2, b2):
    B, C, H, W = x_nchw.shape
    HW = H * W
    Cr = w1.shape[0]
    dtype = x_nchw.dtype
    d_bytes = jnp.dtype(dtype).itemsize

    nb = _group_size(B, C * HW * d_bytes)
    grid = B // nb

    return pl.pallas_call(
        functools.partial(_se_block, inv_hw=1.0 / HW),
        out_shape=jax.ShapeDtypeStruct((B, C, H, W), dtype),
        grid=(grid,),
        in_specs=[
            pl.BlockSpec((nb, C, H, W), lambda i: (i, 0, 0, 0)),
            pl.BlockSpec((Cr, C), lambda i: (0, 0)),
            pl.BlockSpec((1, Cr), lambda i: (0, 0)),
            pl.BlockSpec((C, Cr), lambda i: (0, 0)),
            pl.BlockSpec((1, C), lambda i: (0, 0)),
        ],
        out_specs=pl.BlockSpec((nb, C, H, W), lambda i: (i, 0, 0, 0)),
        compiler_params=pltpu.CompilerParams(
            dimension_semantics=("parallel",),
            vmem_limit_bytes=_VMEM_BYTES,
        ),
        cost_estimate=pl.CostEstimate(
            flops=2 * B * C * HW + 4 * B * C * Cr,
            transcendentals=B * C,
            bytes_accessed=2 * B * C * HW * d_bytes,
        ),
    )(x_nchw, w1, b1.reshape(1, Cr), w2, b2.reshape(1, C))


# speedup vs baseline: 12.1483x; 12.1483x over previous
import functools

import jax
import jax.numpy as jnp
from jax.experimental import pallas as pl
from jax.experimental.pallas import tpu as pltpu

_BLOCK_BYTES_TARGET = 1 * 1024 * 1024
_VMEM_BYTES = 64 * 1024 * 1024

_DN_T = (((1,), (1,)), ((), ()))


def _se_block(x_ref, w1_ref, b1_ref, w2t_ref, b2_ref, o_ref, *, inv_hw):
    x = x_ref[...]
    pooled = jnp.sum(jnp.sum(x.astype(jnp.float32), axis=1), axis=1) * inv_hw
    h = jax.lax.dot_general(pooled, w1_ref[...], _DN_T,
                            preferred_element_type=jnp.float32)
    h = jnp.maximum(h + b1_ref[...], 0.0)
    g = jnp.dot(h, w2t_ref[...], preferred_element_type=jnp.float32)
    g = jax.nn.sigmoid(g + b2_ref[...])
    o_ref[...] = x * g[:, None, None, :].astype(x.dtype)


def _group_size(batch, batch_item_bytes):
    cap = max(1, _BLOCK_BYTES_TARGET // max(batch_item_bytes, 1))
    nb = 1
    for d in range(1, min(batch, cap) + 1):
        if batch % d == 0:
            nb = d
    return nb


def kernel(x_nchw, w1, b1, w2, b2):
    B, C, H, W = x_nchw.shape
    HW = H * W
    Cr = w1.shape[0]
    dtype = x_nchw.dtype
    d_bytes = jnp.dtype(dtype).itemsize

    nb = _group_size(B, C * HW * d_bytes)
    grid = B // nb

    x_nhwc = jnp.transpose(x_nchw, (0, 2, 3, 1))

    out_nhwc = pl.pallas_call(
        functools.partial(_se_block, inv_hw=1.0 / HW),
        out_shape=jax.ShapeDtypeStruct((B, H, W, C), dtype),
        grid=(grid,),
        in_specs=[
            pl.BlockSpec((nb, H, W, C), lambda i: (i, 0, 0, 0)),
            pl.BlockSpec((Cr, C), lambda i: (0, 0)),
            pl.BlockSpec((1, Cr), lambda i: (0, 0)),
            pl.BlockSpec((Cr, C), lambda i: (0, 0)),
            pl.BlockSpec((1, C), lambda i: (0, 0)),
        ],
        out_specs=pl.BlockSpec((nb, H, W, C), lambda i: (i, 0, 0, 0)),
        compiler_params=pltpu.CompilerParams(
            dimension_semantics=("parallel",),
            vmem_limit_bytes=_VMEM_BYTES,
        ),
        cost_estimate=pl.CostEstimate(
            flops=2 * B * C * HW + 4 * B * C * Cr,
            transcendentals=B * C,
            bytes_accessed=2 * B * C * HW * d_bytes,
        ),
    )(x_nhwc, w1, b1.reshape(1, Cr), jnp.transpose(w2), b2.reshape(1, C))

    return jnp.transpose(out_nhwc, (0, 3, 1, 2))


# speedup vs baseline: 19.5953x; 1.6130x over previous
import functools

import jax
import jax.numpy as jnp
from jax.experimental import pallas as pl
from jax.experimental.pallas import tpu as pltpu

_BLOCK_BYTES_TARGET = 4 * 1024 * 1024
_VMEM_BYTES = 64 * 1024 * 1024

_DN_T = (((1,), (1,)), ((), ()))


def _se_block(x_ref, w1_ref, b1_ref, w2t_ref, b2_ref, o_ref, *, inv_hw):
    x = x_ref[...]
    pooled = jnp.sum(jnp.sum(x.astype(jnp.float32), axis=1), axis=1) * inv_hw
    h = jax.lax.dot_general(pooled, w1_ref[...], _DN_T,
                            preferred_element_type=jnp.float32)
    h = jnp.maximum(h + b1_ref[...], 0.0)
    g = jnp.dot(h, w2t_ref[...], preferred_element_type=jnp.float32)
    g = jax.nn.sigmoid(g + b2_ref[...])
    o_ref[...] = x * g[:, None, None, :].astype(x.dtype)


def _group_size(batch, batch_item_bytes):
    cap = max(1, _BLOCK_BYTES_TARGET // max(batch_item_bytes, 1))
    nb = 1
    for d in range(1, min(batch, cap) + 1):
        if batch % d == 0:
            nb = d
    return nb


def kernel(x_nchw, w1, b1, w2, b2):
    B, C, H, W = x_nchw.shape
    HW = H * W
    Cr = w1.shape[0]
    dtype = x_nchw.dtype
    d_bytes = jnp.dtype(dtype).itemsize

    nb = _group_size(B, C * HW * d_bytes)
    grid = B // nb

    x_nhwc = jnp.transpose(x_nchw, (0, 2, 3, 1))

    out_nhwc = pl.pallas_call(
        functools.partial(_se_block, inv_hw=1.0 / HW),
        out_shape=jax.ShapeDtypeStruct((B, H, W, C), dtype),
        grid=(grid,),
        in_specs=[
            pl.BlockSpec((nb, H, W, C), lambda i: (i, 0, 0, 0)),
            pl.BlockSpec((Cr, C), lambda i: (0, 0)),
            pl.BlockSpec((1, Cr), lambda i: (0, 0)),
            pl.BlockSpec((Cr, C), lambda i: (0, 0)),
            pl.BlockSpec((1, C), lambda i: (0, 0)),
        ],
        out_specs=pl.BlockSpec((nb, H, W, C), lambda i: (i, 0, 0, 0)),
        compiler_params=pltpu.CompilerParams(
            dimension_semantics=("parallel",),
            vmem_limit_bytes=_VMEM_BYTES,
        ),
        cost_estimate=pl.CostEstimate(
            flops=2 * B * C * HW + 4 * B * C * Cr,
            transcendentals=B * C,
            bytes_accessed=2 * B * C * HW * d_bytes,
        ),
    )(x_nhwc, w1, b1.reshape(1, Cr), jnp.transpose(w2), b2.reshape(1, C))

    return jnp.transpose(out_nhwc, (0, 3, 1, 2))


# speedup vs baseline: 20.9972x; 1.0715x over previous
import functools

import jax
import jax.numpy as jnp
from jax.experimental import pallas as pl
from jax.experimental.pallas import tpu as pltpu

_BLOCK_BYTES_TARGET = 8 * 1024 * 1024
_VMEM_BYTES = 64 * 1024 * 1024

_DN_T = (((1,), (1,)), ((), ()))


def _se_block(x_ref, w1_ref, b1_ref, w2t_ref, b2_ref, o_ref, *, inv_hw):
    x = x_ref[...]
    pooled = jnp.sum(jnp.sum(x.astype(jnp.float32), axis=1), axis=1) * inv_hw
    h = jax.lax.dot_general(pooled, w1_ref[...], _DN_T,
                            preferred_element_type=jnp.float32)
    h = jnp.maximum(h + b1_ref[...], 0.0)
    g = jnp.dot(h, w2t_ref[...], preferred_element_type=jnp.float32)
    g = jax.nn.sigmoid(g + b2_ref[...])
    o_ref[...] = x * g[:, None, None, :].astype(x.dtype)


def _group_size(batch, batch_item_bytes):
    cap = max(1, _BLOCK_BYTES_TARGET // max(batch_item_bytes, 1))
    nb = 1
    for d in range(1, min(batch, cap) + 1):
        if batch % d == 0:
            nb = d
    return nb


def kernel(x_nchw, w1, b1, w2, b2):
    B, C, H, W = x_nchw.shape
    HW = H * W
    Cr = w1.shape[0]
    dtype = x_nchw.dtype
    d_bytes = jnp.dtype(dtype).itemsize

    nb = _group_size(B, C * HW * d_bytes)
    grid = B // nb

    x_nhwc = jnp.transpose(x_nchw, (0, 2, 3, 1))

    out_nhwc = pl.pallas_call(
        functools.partial(_se_block, inv_hw=1.0 / HW),
        out_shape=jax.ShapeDtypeStruct((B, H, W, C), dtype),
        grid=(grid,),
        in_specs=[
            pl.BlockSpec((nb, H, W, C), lambda i: (i, 0, 0, 0)),
            pl.BlockSpec((Cr, C), lambda i: (0, 0)),
            pl.BlockSpec((1, Cr), lambda i: (0, 0)),
            pl.BlockSpec((Cr, C), lambda i: (0, 0)),
            pl.BlockSpec((1, C), lambda i: (0, 0)),
        ],
        out_specs=pl.BlockSpec((nb, H, W, C), lambda i: (i, 0, 0, 0)),
        compiler_params=pltpu.CompilerParams(
            dimension_semantics=("parallel",),
            vmem_limit_bytes=_VMEM_BYTES,
        ),
        cost_estimate=pl.CostEstimate(
            flops=2 * B * C * HW + 4 * B * C * Cr,
            transcendentals=B * C,
            bytes_accessed=2 * B * C * HW * d_bytes,
        ),
    )(x_nhwc, w1, b1.reshape(1, Cr), jnp.transpose(w2), b2.reshape(1, C))

    return jnp.transpose(out_nhwc, (0, 3, 1, 2))
